# baseline (speedup 1.0000x reference)
import functools

import jax
import jax.numpy as jnp
from jax import lax
from jax.experimental import pallas as pl
from jax.experimental.pallas import tpu as pltpu
from jax.experimental.pallas import tpu_sc as plsc

N = 32768
L = 16
NS = 16
CHUNK = N // NS
NV = CHUNK // L
NG = NV // L

_mesh = plsc.VectorSubcoreMesh(
    core_axis_name="c", subcore_axis_name="s", num_cores=1
)


@functools.partial(
    pl.kernel,
    mesh=_mesh,
    out_type=jax.ShapeDtypeStruct((N,), jnp.float32),
    scratch_types=[
        pltpu.VMEM((CHUNK,), jnp.float32),
        pltpu.VMEM((CHUNK,), jnp.float32),
        pltpu.VMEM((NV,), jnp.float32),
        pltpu.VMEM((NV,), jnp.float32),
        pltpu.VMEM((L,), jnp.float32),
        pltpu.VMEM((NS * L,), jnp.float32),
        pltpu.VMEM_SHARED((NS * L,), jnp.float32),
    ],
    compiler_params=pltpu.CompilerParams(needs_layout_passes=False),
)
def _sc_excl_cumsum(x_hbm, out_hbm, xv, ov, ts, cs, tv, allt, shared):
    sid = lax.axis_index("s")
    base = sid * CHUNK

    pltpu.sync_copy(x_hbm.at[pl.ds(base, CHUNK)], xv)

    lane = lax.iota(jnp.int32, L)
    last_mask = lane == (L - 1)
    last = jnp.full((L,), L - 1, jnp.int32)
    zeros = jnp.zeros((L,), jnp.float32)

    @plsc.parallel_loop(0, NV, unroll=8)
    def _(i):
        v = xv[pl.ds(i * L, L)]
        y = jnp.cumsum(v)
        ov[pl.ds(i * L, L)] = y - v
        plsc.store_scatter(ts, [jnp.full((L,), i, jnp.int32)], y, mask=last_mask)

    @plsc.parallel_loop(0, NG, unroll=8, carry=zeros)
    def acc(j, a):
        return a + ts[pl.ds(j * L, L)]

    tv[...] = jnp.full((L,), jnp.sum(acc), jnp.float32)
    pltpu.sync_copy(tv, shared.at[pl.ds(sid * L, L)])
    plsc.subcore_barrier()
    plsc.subcore_barrier()
    pltpu.sync_copy(shared, allt)

    t_vec = plsc.load_gather(allt, [lane * L])
    offset = jnp.sum(jnp.where(lane < sid, t_vec, zeros))

    def carry_body(j, carry):
        t = ts[pl.ds(j * L, L)]
        yt = jnp.cumsum(t)
        cs[pl.ds(j * L, L)] = (yt - t) + carry
        return carry + yt.at[last].get(mode="promise_in_bounds")

    lax.fori_loop(0, NG, carry_body, jnp.full((L,), offset, jnp.float32))

    @plsc.parallel_loop(0, NV, unroll=8)
    def _(i):
        ov[pl.ds(i * L, L)] += plsc.load_gather(cs, [jnp.full((L,), i, jnp.int32)])

    pltpu.sync_copy(ov, out_hbm.at[pl.ds(base, CHUNK)])


def kernel(input_0):
    return _sc_excl_cumsum(input_0)


# speedup vs baseline: 1.0080x; 1.0080x over previous
import functools

import jax
import jax.numpy as jnp
from jax import lax
from jax.experimental import pallas as pl
from jax.experimental.pallas import tpu as pltpu
from jax.experimental.pallas import tpu_sc as plsc

N = 32768
L = 16
NS = 16
CHUNK = N // NS
NV = CHUNK // L

_mesh = plsc.VectorSubcoreMesh(
    core_axis_name="c", subcore_axis_name="s", num_cores=1
)


@functools.partial(
    pl.kernel,
    mesh=_mesh,
    out_type=jax.ShapeDtypeStruct((N,), jnp.float32),
    scratch_types=[
        pltpu.VMEM((CHUNK,), jnp.float32),
        pltpu.VMEM((CHUNK,), jnp.float32),
        pltpu.VMEM((L,), jnp.float32),
        pltpu.VMEM((NS * L,), jnp.float32),
        pltpu.VMEM_SHARED((NS * L,), jnp.float32),
    ],
    compiler_params=pltpu.CompilerParams(needs_layout_passes=False),
)
def _sc_excl_cumsum(x_hbm, out_hbm, xv, ov, tv, allt, shared):
    sid = lax.axis_index("s")
    base = sid * CHUNK

    pltpu.sync_copy(x_hbm.at[pl.ds(base, CHUNK)], xv)

    @plsc.parallel_loop(0, NV, unroll=8, carry=jnp.zeros((L,), jnp.float32))
    def acc(i, a):
        return a + xv[pl.ds(i * L, L)]

    total = jnp.sum(acc)

    tv[...] = jnp.full((L,), total, jnp.float32)
    pltpu.sync_copy(tv, shared.at[pl.ds(sid * L, L)])
    plsc.subcore_barrier()
    plsc.subcore_barrier()
    pltpu.sync_copy(shared, allt)

    lane = lax.iota(jnp.int32, L)
    t_vec = plsc.load_gather(allt, [lane * L])
    offset = jnp.sum(jnp.where(lane < sid, t_vec, jnp.zeros((L,), jnp.float32)))

    last = jnp.full((L,), L - 1, jnp.int32)

    @plsc.parallel_loop(
        0, NV, unroll=8, carry=jnp.full((L,), offset, jnp.float32)
    )
    def _(i, carry):
        v = xv[pl.ds(i * L, L)]
        y = jnp.cumsum(v)
        ov[pl.ds(i * L, L)] = (y - v) + carry
        return carry + y.at[last].get(mode="promise_in_bounds")

    pltpu.sync_copy(ov, out_hbm.at[pl.ds(base, CHUNK)])


def kernel(input_0):
    return _sc_excl_cumsum(input_0)


# speedup vs baseline: 1.0153x; 1.0073x over previous
import functools

import jax
import jax.numpy as jnp
from jax import lax
from jax.experimental import pallas as pl
from jax.experimental.pallas import tpu as pltpu
from jax.experimental.pallas import tpu_sc as plsc

N = 32768
L = 16
NS = 16
CHUNK = N // NS
NV = CHUNK // L

_mesh = plsc.VectorSubcoreMesh(
    core_axis_name="c", subcore_axis_name="s", num_cores=1
)


@functools.partial(
    pl.kernel,
    mesh=_mesh,
    out_type=jax.ShapeDtypeStruct((N,), jnp.float32),
    scratch_types=[
        pltpu.VMEM((CHUNK,), jnp.float32),
        pltpu.VMEM((CHUNK,), jnp.float32),
        pltpu.VMEM_SHARED((NS * L,), jnp.float32),
    ],
    compiler_params=pltpu.CompilerParams(needs_layout_passes=False),
)
def _sc_excl_cumsum(x_hbm, out_hbm, xv, ov, shared):
    sid = lax.axis_index("s")
    base = sid * CHUNK

    pltpu.sync_copy(x_hbm.at[pl.ds(base, CHUNK)], xv)

    @plsc.parallel_loop(0, NV, unroll=8, carry=jnp.zeros((L,), jnp.float32))
    def acc(i, a):
        return a + xv[pl.ds(i * L, L)]

    total = jnp.sum(acc)

    ov[pl.ds(0, L)] = jnp.full((L,), total, jnp.float32)
    pltpu.sync_copy(ov.at[pl.ds(0, L)], shared.at[pl.ds(sid * L, L)])
    plsc.subcore_barrier()
    plsc.subcore_barrier()
    pltpu.sync_copy(shared, ov.at[pl.ds(0, NS * L)])

    lane = lax.iota(jnp.int32, L)
    t_vec = plsc.load_gather(ov, [lane * L])
    offset = jnp.sum(jnp.where(lane < sid, t_vec, jnp.zeros((L,), jnp.float32)))

    last = jnp.full((L,), L - 1, jnp.int32)

    @plsc.parallel_loop(
        0, NV, unroll=8, carry=jnp.full((L,), offset, jnp.float32)
    )
    def _(i, carry):
        v = xv[pl.ds(i * L, L)]
        y = jnp.cumsum(v)
        ov[pl.ds(i * L, L)] = (y - v) + carry
        return carry + y.at[last].get(mode="promise_in_bounds")

    pltpu.sync_copy(ov, out_hbm.at[pl.ds(base, CHUNK)])


def kernel(input_0):
    return _sc_excl_cumsum(input_0)
